# Initial kernel scaffold; baseline (speedup 1.0000x reference)
import jax
import jax.numpy as jnp
from jax.experimental import pallas as pl


def kernel(action_pred_logits, pred_boxes, pred_boxes_mask):
    raise NotImplementedError("write your pallas kernel here")


# speedup vs baseline: 8.0527x; 8.0527x over previous
import functools

import jax
import jax.numpy as jnp
from jax import lax
from jax.experimental import pallas as pl
from jax.experimental.pallas import tpu as pltpu
from jax.experimental.pallas import tpu_sc as plsc

B, Q, C = 16, 20000, 80
K_TOP = 100
NC, NS, L = 2, 16, 16
NW = NC * NS
ROWS_W = B * Q // NW
WIN = 400
NWIN = ROWS_W // WIN
CAP = 128
NEG = -3.0e38

_mesh = plsc.VectorSubcoreMesh(core_axis_name="c", subcore_axis_name="s")


def _wid():
    return lax.axis_index("s") * NC + lax.axis_index("c")


def _mono(v):
    u = lax.bitcast_convert_type(v, jnp.uint32)
    flip = jnp.where(v < 0.0, jnp.uint32(0xFFFFFFFF), jnp.uint32(0x80000000))
    return u ^ flip


def _lanesum(v_i32):
    return jnp.sum(v_i32, axis=0)


def _bload(ref, j):
    return plsc.load_gather(ref, [jnp.zeros((16,), jnp.int32) + j])


def _radix_select_u(read_vreg, nv, k, hist, suf):
    lanes = lax.iota(jnp.int32, 16)
    ones = jnp.ones((16,), jnp.int32)
    zeros = jnp.zeros((16,), jnp.int32)
    pref = jnp.uint32(0)
    kk = jnp.int32(k)
    for p in range(4):
        sh = 24 - 8 * p

        def zbody(h, _):
            hist[pl.ds(h * 16, 16)] = zeros
            return 0

        lax.fori_loop(0, 256, zbody, 0)

        def hbody(vi, _, _sh=sh, _p=p, _pref=pref):
            u = read_vreg(vi)
            bin_ = ((u >> jnp.uint32(_sh)) & jnp.uint32(255)).astype(jnp.int32)
            idx = bin_ * 16 + lanes
            if _p == 0:
                plsc.addupdate_scatter(hist, [idx], ones)
            else:
                m = (u >> jnp.uint32(_sh + 8)) == (_pref >> jnp.uint32(_sh + 8))
                plsc.addupdate_scatter(hist, [idx], ones, mask=m)
            return 0

        lax.fori_loop(0, nv, hbody, 0)

        def sbody(i, acc):
            b_ = 255 - i
            acc = acc + hist[pl.ds(b_ * 16, 16)]
            suf[pl.ds(b_ * 16, 16)] = acc
            return acc

        lax.fori_loop(0, 256, sbody, zeros)

        def bbody(_, lohi):
            lo, hi = lohi
            mid = (lo + hi) // 2
            smid = _lanesum(suf[pl.ds(mid * 16, 16)])
            ok = smid >= kk
            return (jnp.where(ok, mid, lo), jnp.where(ok, hi, mid))

        lo, _hi = lax.fori_loop(0, 8, bbody, (jnp.int32(0), jnp.int32(256)))
        t = lo
        nxt = jnp.minimum(t + 1, 255)
        s_above = jnp.where(t >= 255, jnp.int32(0),
                            _lanesum(suf[pl.ds(nxt * 16, 16)]))
        kk = kk - s_above
        pref = pref | (t.astype(jnp.uint32) << jnp.uint32(sh))
    return pref


@functools.partial(
    pl.kernel, mesh=_mesh,
    compiler_params=pltpu.CompilerParams(needs_layout_passes=False),
    out_type=jax.ShapeDtypeStruct((B * Q,), jnp.float32),
    scratch_types=[
        pltpu.VMEM((WIN * C,), jnp.float32),
        pltpu.VMEM((ROWS_W,), jnp.int32),
        pltpu.VMEM((ROWS_W,), jnp.float32),
    ],
)
def _rowmax_k(logits_hbm, mask_hbm, rm_hbm, buf, maskv, rmv):
    w = _wid()
    base = w * ROWS_W
    pltpu.sync_copy(mask_hbm.at[pl.ds(base, ROWS_W)], maskv)

    lanes = lax.iota(jnp.int32, 16)
    lanes_row = lanes * C

    def win_body(wi, _):
        pltpu.sync_copy(logits_hbm.at[pl.ds((base + wi * WIN) * C, WIN * C)],
                        buf)

        def grp_body(g, _):
            fbase = g * (16 * C) + lanes_row
            accs = []
            for c in range(4):
                accs.append(plsc.load_gather(buf, [fbase + c]))
            for c in range(4, C):
                accs[c % 4] = jnp.maximum(
                    accs[c % 4], plsc.load_gather(buf, [fbase + c]))
            acc = jnp.maximum(jnp.maximum(accs[0], accs[1]),
                              jnp.maximum(accs[2], accs[3]))
            rr = wi * WIN + g * 16
            mv = maskv[pl.ds(rr, 16)]
            rmv[pl.ds(rr, 16)] = jnp.where(mv != 0, NEG, acc)
            return 0

        lax.fori_loop(0, WIN // 16, grp_body, 0)
        return 0

    lax.fori_loop(0, NWIN, win_body, 0)
    pltpu.sync_copy(rmv, rm_hbm.at[pl.ds(base, ROWS_W)])


@functools.partial(
    pl.kernel, mesh=_mesh,
    compiler_params=pltpu.CompilerParams(needs_layout_passes=False),
    out_type=(
        jax.ShapeDtypeStruct((B, 128), jnp.float32),
        jax.ShapeDtypeStruct((B, 128), jnp.int32),
        jax.ShapeDtypeStruct((B, 512), jnp.float32),
    ),
    scratch_types=[
        pltpu.VMEM((Q,), jnp.float32),
        pltpu.VMEM((256 * 16,), jnp.int32),
        pltpu.VMEM((256 * 16,), jnp.int32),
        pltpu.VMEM((CAP + 16,), jnp.int32),
        pltpu.VMEM((CAP * C,), jnp.float32),
        pltpu.VMEM((128,), jnp.float32),
        pltpu.VMEM((128,), jnp.int32),
        pltpu.VMEM((128,), jnp.float32),
        pltpu.VMEM((128,), jnp.int32),
        pltpu.VMEM((128,), jnp.int32),
        pltpu.VMEM((Q * 4,), jnp.float32),
        pltpu.VMEM((128,), jnp.float32),
        pltpu.VMEM((128,), jnp.int32),
        pltpu.VMEM((512,), jnp.float32),
        pltpu.SemaphoreType.DMA,
        pltpu.SemaphoreType.DMA,
    ],
)
def _select_k(logits_hbm, rm_hbm, boxes_hbm, scores_hbm, labels_hbm,
              oboxes_hbm, rmv, hist, suf, rowids, cand, vals2, idx2,
              svals, sidx, srow, boxv, osc, olab, obox, sem, bsem):
    w = _wid()

    @pl.when(w < B)
    def _():
        b = w
        bq = b * Q
        lanes = lax.iota(jnp.int32, 16)
        zeros = jnp.zeros((16,), jnp.int32)
        negv = jnp.full((16,), NEG, jnp.float32)

        pltpu.sync_copy(rm_hbm.at[pl.ds(bq, Q)], rmv)
        box_cp = pltpu.async_copy(boxes_hbm.at[pl.ds(bq * 4, Q * 4)], boxv,
                                  bsem)

        def rd_rm(vi):
            return _mono(rmv[pl.ds(vi * 16, 16)])

        t_row = _radix_select_u(rd_rm, Q // 16, K_TOP, hist, suf)

        for h in range(CAP // 16 + 1):
            rowids[pl.ds(h * 16, 16)] = bq + h * 16 + lanes

        def cbody(vi, off):
            v = rmv[pl.ds(vi * 16, 16)]
            u = _mono(v)
            m = (u >= t_row) & (v > NEG)
            cnt = plsc.all_reduce_population_count(m)

            @pl.when(jnp.any(m))
            def _():
                pos = off + plsc.cumsum(m.astype(jnp.int32)) - 1
                pos = jnp.minimum(pos, CAP - 1)
                plsc.store_scatter(rowids, [pos], bq + vi * 16 + lanes, mask=m)

            return off + cnt

        off = lax.fori_loop(0, Q // 16, cbody, zeros)
        nrows = jnp.minimum(jnp.max(off, axis=0), jnp.int32(CAP))

        def fire_body(j, _):
            r = rowids[pl.ds(j, 16)][0]
            pltpu.async_copy(logits_hbm.at[pl.ds(r * C, C)],
                             cand.at[pl.ds(j * C, C)], sem)
            return 0

        lax.fori_loop(0, CAP, fire_body, 0)

        def drain_body(j, _):
            pltpu.make_async_copy(logits_hbm.at[pl.ds(0, C)],
                                  cand.at[pl.ds(j * C, C)], sem).wait()
            return 0

        lax.fori_loop(0, CAP, drain_body, 0)

        def ibody(j, _):
            rloc = _bload(rowids, j) - bq
            rmj = plsc.load_gather(rmv, [rloc])
            keep = (j < nrows) & (rmj > NEG)
            for i in range(5):
                v = cand[pl.ds(j * C + i * 16, 16)]
                cand[pl.ds(j * C + i * 16, 16)] = jnp.where(keep, v, negv)
            return 0

        lax.fori_loop(0, CAP, ibody, 0)

        def rd_cand(vi):
            return _mono(cand[pl.ds(vi * 16, 16)])

        t_el = _radix_select_u(rd_cand, CAP * 5, K_TOP, hist, suf)

        for h in range(8):
            vals2[pl.ds(h * 16, 16)] = negv
            idx2[pl.ds(h * 16, 16)] = zeros + jnp.int32(1 << 28)

        def c2body(j, off2):
            rloc = _bload(rowids, j) - bq
            for i in range(5):
                v = cand[pl.ds(j * C + i * 16, 16)]
                u = _mono(v)
                m = u >= t_el
                cnt = plsc.all_reduce_population_count(m)

                @pl.when(jnp.any(m))
                def _(off2=off2, v=v, m=m, rloc=rloc, i=i):
                    pos = off2 + plsc.cumsum(m.astype(jnp.int32)) - 1
                    pos = jnp.minimum(pos, 127)
                    plsc.store_scatter(vals2, [pos], v, mask=m)
                    plsc.store_scatter(idx2, [pos],
                                      rloc * C + i * 16 + lanes, mask=m)

                off2 = off2 + cnt
            return off2

        lax.fori_loop(0, CAP, c2body, zeros)

        for h in range(8):
            svals[pl.ds(h * 16, 16)] = negv
            sidx[pl.ds(h * 16, 16)] = zeros

        lane0 = lanes == 0

        def rbody(j, _):
            vj = _bload(vals2, j)
            ij = _bload(idx2, j)
            acc = zeros
            for h in range(8):
                vv = vals2[pl.ds(h * 16, 16)]
                iv = idx2[pl.ds(h * 16, 16)]
                gt = (vv > vj) | ((vv == vj) & (iv < ij))
                acc = acc + gt.astype(jnp.int32)
            rank = jnp.minimum(_lanesum(acc), jnp.int32(127))
            rankv = jnp.zeros((16,), jnp.int32) + rank
            plsc.store_scatter(svals, [rankv], vj, mask=lane0)
            plsc.store_scatter(sidx, [rankv], ij, mask=lane0)
            return 0

        lax.fori_loop(0, 128, rbody, 0)

        for h in range(8):
            v = svals[pl.ds(h * 16, 16)]
            osc[pl.ds(h * 16, 16)] = 1.0 / (1.0 + jnp.exp(-v))
            iv = sidx[pl.ds(h * 16, 16)]
            olab[pl.ds(h * 16, 16)] = lax.rem(iv, jnp.int32(C))
            srow[pl.ds(h * 16, 16)] = jnp.minimum(lax.div(iv, jnp.int32(C)),
                                                  jnp.int32(Q - 1))

        box_cp.wait()

        def gbody(g, _):
            rows = plsc.load_gather(srow, [g * 4 + (lanes >> 2)])
            obox[pl.ds(g * 16, 16)] = plsc.load_gather(
                boxv, [rows * 4 + (lanes & 3)])
            return 0

        lax.fori_loop(0, 32, gbody, 0)

        pltpu.sync_copy(osc, scores_hbm.at[b])
        pltpu.sync_copy(olab, labels_hbm.at[b])
        pltpu.sync_copy(obox, oboxes_hbm.at[b])


def kernel(action_pred_logits, pred_boxes, pred_boxes_mask):
    logits1d = action_pred_logits.reshape(B * Q * C)
    mask_i = pred_boxes_mask.reshape(B * Q).astype(jnp.int32)
    boxes1d = pred_boxes.reshape(B * Q * 4)
    rm = _rowmax_k(logits1d, mask_i)
    scores_p, labels_p, boxes_p = _select_k(logits1d, rm, boxes1d)
    return (scores_p[:, :K_TOP], labels_p[:, :K_TOP],
            boxes_p.reshape(B, 128, 4)[:, :K_TOP, :])


# speedup vs baseline: 18.3972x; 2.2846x over previous
import functools

import jax
import jax.numpy as jnp
from jax import lax
from jax.experimental import pallas as pl
from jax.experimental.pallas import tpu as pltpu
from jax.experimental.pallas import tpu_sc as plsc

B, Q, C = 16, 20000, 80
K_TOP = 100
NC, NS, L = 2, 16, 16
NW = NC * NS
ROWS_W = B * Q // NW
WIN = 400
NWIN = ROWS_W // WIN
CAP = 128
NEG = -3.0e38

_mesh = plsc.VectorSubcoreMesh(core_axis_name="c", subcore_axis_name="s")


def _wid():
    return lax.axis_index("s") * NC + lax.axis_index("c")


def _mono(v):
    u = lax.bitcast_convert_type(v, jnp.uint32)
    flip = jnp.where(v < 0.0, jnp.uint32(0xFFFFFFFF), jnp.uint32(0x80000000))
    return u ^ flip


def _lanesum(v_i32):
    return jnp.sum(v_i32, axis=0)


def _bload(ref, j):
    return plsc.load_gather(ref, [jnp.zeros((16,), jnp.int32) + j])


def _radix_select_u(read_vreg, nv, k, hist, suf):
    lanes = lax.iota(jnp.int32, 16)
    ones = jnp.ones((16,), jnp.int32)
    zeros = jnp.zeros((16,), jnp.int32)
    pref = jnp.uint32(0)
    kk = jnp.int32(k)
    for p in range(4):
        sh = 24 - 8 * p

        def zbody(h, _):
            hist[pl.ds(h * 16, 16)] = zeros
            return 0

        lax.fori_loop(0, 256, zbody, 0)

        def hbody(vi, _, _sh=sh, _p=p, _pref=pref):
            u = read_vreg(vi)
            bin_ = ((u >> jnp.uint32(_sh)) & jnp.uint32(255)).astype(jnp.int32)
            idx = bin_ * 16 + lanes
            if _p == 0:
                plsc.addupdate_scatter(hist, [idx], ones)
            else:
                m = (u >> jnp.uint32(_sh + 8)) == (_pref >> jnp.uint32(_sh + 8))
                plsc.addupdate_scatter(hist, [idx], ones, mask=m)
            return 0

        lax.fori_loop(0, nv, hbody, 0)

        def sbody(i, acc):
            b_ = 255 - i
            acc = acc + hist[pl.ds(b_ * 16, 16)]
            suf[pl.ds(b_ * 16, 16)] = acc
            return acc

        lax.fori_loop(0, 256, sbody, zeros)

        def bbody(_, lohi):
            lo, hi = lohi
            mid = (lo + hi) // 2
            smid = _lanesum(suf[pl.ds(mid * 16, 16)])
            ok = smid >= kk
            return (jnp.where(ok, mid, lo), jnp.where(ok, hi, mid))

        lo, _hi = lax.fori_loop(0, 8, bbody, (jnp.int32(0), jnp.int32(256)))
        t = lo
        nxt = jnp.minimum(t + 1, 255)
        s_above = jnp.where(t >= 255, jnp.int32(0),
                            _lanesum(suf[pl.ds(nxt * 16, 16)]))
        kk = kk - s_above
        pref = pref | (t.astype(jnp.uint32) << jnp.uint32(sh))
    return pref


@functools.partial(
    pl.kernel, mesh=_mesh,
    compiler_params=pltpu.CompilerParams(needs_layout_passes=False),
    out_type=jax.ShapeDtypeStruct((B * Q,), jnp.float32),
    scratch_types=[
        pltpu.VMEM((WIN, C), jnp.float32),
        pltpu.VMEM((WIN, C), jnp.float32),
        pltpu.VMEM((ROWS_W,), jnp.int32),
        pltpu.VMEM((ROWS_W,), jnp.float32),
        pltpu.SemaphoreType.DMA,
        pltpu.SemaphoreType.DMA,
    ],
)
def _rowmax_k(logits_hbm, mask_hbm, rm_hbm, buf0, buf1, maskv, rmv, s0, s1):
    w = _wid()
    b = w // 2
    row0 = (w % 2) * ROWS_W
    base = w * ROWS_W
    pltpu.sync_copy(mask_hbm.at[pl.ds(base, ROWS_W)], maskv)

    lanes = lax.iota(jnp.int32, 16)
    lane0 = lanes == 0
    zeros = jnp.zeros((16,), jnp.int32)
    fzeros = jnp.zeros((16,), jnp.float32)

    def do_window(wi, buf):
        def rbody(rr, _):
            v0 = buf[rr, pl.ds(0, 16)]
            v1 = buf[rr, pl.ds(16, 16)]
            v2 = buf[rr, pl.ds(32, 16)]
            v3 = buf[rr, pl.ds(48, 16)]
            v4 = buf[rr, pl.ds(64, 16)]
            acc = jnp.maximum(jnp.maximum(jnp.maximum(v0, v1),
                                          jnp.maximum(v2, v3)), v4)
            rmax = jnp.max(acc)
            plsc.store_scatter(rmv, [zeros + (wi * WIN + rr)],
                               fzeros + rmax, mask=lane0)
            return 0

        lax.fori_loop(0, WIN, rbody, 0)

    pltpu.async_copy(logits_hbm.at[b, pl.ds(row0, WIN)], buf0, s0)

    def pair_body(g2, _):
        for sub, bufa, sema, bufb, semb in ((0, buf0, s0, buf1, s1),
                                            (1, buf1, s1, buf0, s0)):
            wi = g2 * 2 + sub
            pltpu.make_async_copy(logits_hbm.at[b, pl.ds(row0, WIN)],
                                  bufa, sema).wait()
            pltpu.async_copy(
                logits_hbm.at[b, pl.ds(row0 + (wi + 1) * WIN, WIN)],
                bufb, semb)
            do_window(wi, bufa)
        return 0

    lax.fori_loop(0, (NWIN - 1) // 2, pair_body, 0)
    pltpu.make_async_copy(logits_hbm.at[b, pl.ds(row0, WIN)],
                          buf0, s0).wait()
    do_window(NWIN - 1, buf0)

    negv = jnp.full((16,), NEG, jnp.float32)

    def mbody(vi, _):
        mv = maskv[pl.ds(vi * 16, 16)]
        rv = rmv[pl.ds(vi * 16, 16)]
        rmv[pl.ds(vi * 16, 16)] = jnp.where(mv != 0, negv, rv)
        return 0

    lax.fori_loop(0, ROWS_W // 16, mbody, 0)
    pltpu.sync_copy(rmv, rm_hbm.at[pl.ds(base, ROWS_W)])


@functools.partial(
    pl.kernel, mesh=_mesh,
    compiler_params=pltpu.CompilerParams(needs_layout_passes=False),
    out_type=(
        jax.ShapeDtypeStruct((B, 128), jnp.float32),
        jax.ShapeDtypeStruct((B, 128), jnp.int32),
        jax.ShapeDtypeStruct((B, 128, 4), jnp.float32),
    ),
    scratch_types=[
        pltpu.VMEM((Q,), jnp.float32),
        pltpu.VMEM((256 * 16,), jnp.int32),
        pltpu.VMEM((256 * 16,), jnp.int32),
        pltpu.VMEM((CAP + 16,), jnp.int32),
        pltpu.VMEM((CAP, C), jnp.float32),
        pltpu.VMEM((CAP * C,), jnp.float32),
        pltpu.VMEM((128,), jnp.float32),
        pltpu.VMEM((128,), jnp.int32),
        pltpu.VMEM((128,), jnp.float32),
        pltpu.VMEM((128,), jnp.int32),
        pltpu.VMEM((128 + 16,), jnp.int32),
        pltpu.VMEM((128, 4), jnp.float32),
        pltpu.VMEM((128,), jnp.float32),
        pltpu.VMEM((128,), jnp.int32),
        pltpu.SemaphoreType.DMA,
        pltpu.SemaphoreType.DMA,
    ],
)
def _select_k(logits_hbm, rm_hbm, boxes_hbm, scores_hbm, labels_hbm,
              oboxes_hbm, rmv, hist, suf, rowids, cand2, cand, vals2, idx2,
              svals, sidx, srow, obox2, osc, olab, sem, bsem):
    w = _wid()

    @pl.when(w < B)
    def _():
        b = w
        bq = b * Q
        lanes = lax.iota(jnp.int32, 16)
        zeros = jnp.zeros((16,), jnp.int32)
        negv = jnp.full((16,), NEG, jnp.float32)

        pltpu.sync_copy(rm_hbm.at[pl.ds(bq, Q)], rmv)

        def rd_rm(vi):
            return _mono(rmv[pl.ds(vi * 16, 16)])

        t_row = _radix_select_u(rd_rm, Q // 16, K_TOP, hist, suf)

        for h in range(CAP // 16 + 1):
            rowids[pl.ds(h * 16, 16)] = h * 16 + lanes

        def cbody(vi, off):
            v = rmv[pl.ds(vi * 16, 16)]
            u = _mono(v)
            m = (u >= t_row) & (v > NEG)
            cnt = plsc.all_reduce_population_count(m)

            @pl.when(jnp.any(m))
            def _():
                pos = off + plsc.cumsum(m.astype(jnp.int32)) - 1
                pos = jnp.minimum(pos, CAP - 1)
                plsc.store_scatter(rowids, [pos], vi * 16 + lanes, mask=m)

            return off + cnt

        off = lax.fori_loop(0, Q // 16, cbody, zeros)
        nrows = jnp.minimum(jnp.max(off, axis=0), jnp.int32(CAP))

        def fire_body(j, _):
            r = rowids[pl.ds(j, 16)][0]
            pltpu.async_copy(logits_hbm.at[b, pl.ds(r, 1)],
                             cand2.at[pl.ds(j, 1)], sem)
            return 0

        lax.fori_loop(0, CAP, fire_body, 0)

        def drain_body(j, _):
            pltpu.make_async_copy(logits_hbm.at[b, pl.ds(0, 1)],
                                  cand2.at[pl.ds(j, 1)], sem).wait()
            return 0

        lax.fori_loop(0, CAP, drain_body, 0)

        def ibody(j, _):
            rloc = _bload(rowids, j)
            rmj = plsc.load_gather(rmv, [rloc])
            keep = (j < nrows) & (rmj > NEG)
            for i in range(5):
                v = cand2[j, pl.ds(i * 16, 16)]
                cand[pl.ds(j * C + i * 16, 16)] = jnp.where(keep, v, negv)
            return 0

        lax.fori_loop(0, CAP, ibody, 0)

        def rd_cand(vi):
            return _mono(cand[pl.ds(vi * 16, 16)])

        t_el = _radix_select_u(rd_cand, CAP * 5, K_TOP, hist, suf)

        for h in range(8):
            vals2[pl.ds(h * 16, 16)] = negv
            idx2[pl.ds(h * 16, 16)] = zeros + jnp.int32(1 << 28)

        def c2body(j, off2):
            rloc = _bload(rowids, j)
            for i in range(5):
                v = cand[pl.ds(j * C + i * 16, 16)]
                u = _mono(v)
                m = u >= t_el
                cnt = plsc.all_reduce_population_count(m)

                @pl.when(jnp.any(m))
                def _(off2=off2, v=v, m=m, rloc=rloc, i=i):
                    pos = off2 + plsc.cumsum(m.astype(jnp.int32)) - 1
                    pos = jnp.minimum(pos, 127)
                    plsc.store_scatter(vals2, [pos], v, mask=m)
                    plsc.store_scatter(idx2, [pos],
                                      rloc * C + i * 16 + lanes, mask=m)

                off2 = off2 + cnt
            return off2

        lax.fori_loop(0, CAP, c2body, zeros)

        for h in range(8):
            svals[pl.ds(h * 16, 16)] = negv
            sidx[pl.ds(h * 16, 16)] = zeros

        lane0 = lanes == 0

        def rbody(j, _):
            vj = _bload(vals2, j)
            ij = _bload(idx2, j)
            acc = zeros
            for h in range(8):
                vv = vals2[pl.ds(h * 16, 16)]
                iv = idx2[pl.ds(h * 16, 16)]
                gt = (vv > vj) | ((vv == vj) & (iv < ij))
                acc = acc + gt.astype(jnp.int32)
            rank = jnp.minimum(_lanesum(acc), jnp.int32(127))
            rankv = jnp.zeros((16,), jnp.int32) + rank
            plsc.store_scatter(svals, [rankv], vj, mask=lane0)
            plsc.store_scatter(sidx, [rankv], ij, mask=lane0)
            return 0

        lax.fori_loop(0, 128, rbody, 0)

        for h in range(8):
            v = svals[pl.ds(h * 16, 16)]
            osc[pl.ds(h * 16, 16)] = 1.0 / (1.0 + jnp.exp(-v))
            iv = sidx[pl.ds(h * 16, 16)]
            olab[pl.ds(h * 16, 16)] = lax.rem(iv, jnp.int32(C))
            srow[pl.ds(h * 16, 16)] = jnp.minimum(lax.div(iv, jnp.int32(C)),
                                                  jnp.int32(Q - 1))

        def bfire(k, _):
            r = srow[pl.ds(k, 16)][0]
            pltpu.async_copy(boxes_hbm.at[b, pl.ds(r, 1)],
                             obox2.at[pl.ds(k, 1)], bsem)
            return 0

        lax.fori_loop(0, 128, bfire, 0)

        def bdrain(k, _):
            pltpu.make_async_copy(boxes_hbm.at[b, pl.ds(0, 1)],
                                  obox2.at[pl.ds(k, 1)], bsem).wait()
            return 0

        lax.fori_loop(0, 128, bdrain, 0)

        pltpu.sync_copy(osc, scores_hbm.at[b])
        pltpu.sync_copy(olab, labels_hbm.at[b])
        pltpu.sync_copy(obox2, oboxes_hbm.at[b])


def kernel(action_pred_logits, pred_boxes, pred_boxes_mask):
    mask_i = pred_boxes_mask.reshape(B * Q).astype(jnp.int32)
    rm = _rowmax_k(action_pred_logits, mask_i)
    scores_p, labels_p, boxes_p = _select_k(action_pred_logits, rm,
                                            pred_boxes)
    return (scores_p[:, :K_TOP], labels_p[:, :K_TOP],
            boxes_p[:, :K_TOP, :])
